# baseline (speedup 1.0000x reference)
import functools

import jax
import jax.numpy as jnp
from jax import lax
from jax.experimental import pallas as pl
from jax.experimental.pallas import tpu as pltpu
from jax.experimental.pallas import tpu_sc as plsc

N = 16384
D = 64
NUM_CORES = 2
NUM_SUBCORES = 16
NUM_WORKERS = NUM_CORES * NUM_SUBCORES
B_PER_W = N // NUM_WORKERS
LANES = 16
N_CHUNKS = B_PER_W // LANES
INT_PO = 1
INT_NE = 2


@functools.cache
def _build_sc_embed():
    mesh = plsc.VectorSubcoreMesh(
        core_axis_name="c",
        subcore_axis_name="s",
        num_cores=NUM_CORES,
        num_subcores=NUM_SUBCORES,
    )
    return pl.kernel(
        _sc_embed_body,
        out_type=jax.ShapeDtypeStruct((N, D), jnp.float32),
        mesh=mesh,
        compiler_params=pltpu.CompilerParams(
            use_tc_tiling_on_sc=True, needs_layout_passes=False
        ),
        scratch_types=[
            pltpu.VMEM((B_PER_W,), jnp.int32),
            pltpu.VMEM((B_PER_W,), jnp.float32),
            pltpu.VMEM((B_PER_W, D), jnp.float32),
            pltpu.SemaphoreType.DMA,
        ],
    )


def _sc_embed_body(
    types_hbm, args_hbm, table_hbm, out_hbm, idx_v, args_v, rows_v, sem
):
    wid = lax.axis_index("s") * NUM_CORES + lax.axis_index("c")
    base = wid * B_PER_W

    pltpu.sync_copy(types_hbm.at[pl.ds(base, B_PER_W)], idx_v)

    lane = lax.iota(jnp.int32, LANES)

    def chunk(i, carry):
        b16 = pl.multiple_of(i * LANES, LANES)
        t = idx_v[pl.ds(b16, LANES)]

        for l in range(LANES):
            row = jnp.max(jnp.where(lane == l, t, 0))
            pltpu.async_copy(
                table_hbm.at[pl.ds(row, 1)],
                rows_v.at[pl.ds(b16 + l, 1)],
                sem,
            )
        return carry

    lax.fori_loop(0, N_CHUNKS, chunk, 0)

    pltpu.sync_copy(args_hbm.at[pl.ds(base, B_PER_W)], args_v)

    pltpu.make_async_copy(
        table_hbm.at[pl.ds(0, B_PER_W)], rows_v, sem
    ).wait()

    col63 = lane * 0 + (D - 1)

    def fix(i, carry):
        b16 = pl.multiple_of(i * LANES, LANES)
        t = idx_v[pl.ds(b16, LANES)]
        a = args_v[pl.ds(b16, LANES)]
        m = (t == INT_PO) | (t == INT_NE)
        plsc.store_scatter(rows_v, [b16 + lane, col63], a, mask=m)
        return carry

    lax.fori_loop(0, N_CHUNKS, fix, 0)

    pltpu.sync_copy(rows_v, out_hbm.at[pl.ds(base, B_PER_W)])


def kernel(node_types, node_args, table):
    out = _build_sc_embed()(node_types, node_args, table)
    return out.reshape(N, 1, D)


# speedup vs baseline: 1.0027x; 1.0027x over previous
import functools

import jax
import jax.numpy as jnp
from jax import lax
from jax.experimental import pallas as pl
from jax.experimental.pallas import tpu as pltpu
from jax.experimental.pallas import tpu_sc as plsc

N = 16384
D = 64
NUM_CORES = 2
NUM_SUBCORES = 16
NUM_WORKERS = NUM_CORES * NUM_SUBCORES
B_PER_W = N // NUM_WORKERS
LANES = 16
N_CHUNKS = B_PER_W // LANES
INT_PO = 1
INT_NE = 2


@functools.cache
def _build_sc_embed():
    mesh = plsc.VectorSubcoreMesh(
        core_axis_name="c",
        subcore_axis_name="s",
        num_cores=NUM_CORES,
        num_subcores=NUM_SUBCORES,
    )
    return pl.kernel(
        _sc_embed_body,
        out_type=jax.ShapeDtypeStruct((N, D), jnp.float32),
        mesh=mesh,
        compiler_params=pltpu.CompilerParams(
            use_tc_tiling_on_sc=True, needs_layout_passes=False
        ),
        scratch_types=[
            pltpu.VMEM((B_PER_W,), jnp.int32),
            pltpu.VMEM((B_PER_W,), jnp.float32),
            pltpu.VMEM((B_PER_W, D), jnp.float32),
            pltpu.SemaphoreType.DMA,
        ],
    )


def _sc_embed_body(
    types_hbm, args_hbm, table_hbm, out_hbm, idx_v, args_v, rows_v, sem
):
    wid = lax.axis_index("s") * NUM_CORES + lax.axis_index("c")
    base = wid * B_PER_W

    pltpu.sync_copy(types_hbm.at[pl.ds(base, B_PER_W)], idx_v)

    lane = lax.iota(jnp.int32, LANES)

    def chunk(i, carry):
        b16 = pl.multiple_of(i * LANES, LANES)
        t = idx_v[pl.ds(b16, LANES)]

        for l in range(LANES):
            row = jnp.max(jnp.where(lane == l, t, 0))
            pltpu.async_copy(
                table_hbm.at[pl.ds(row, 1)],
                rows_v.at[pl.ds(b16 + l, 1)],
                sem,
            )
        return carry

    plsc.parallel_loop(0, N_CHUNKS, 1)(lambda i: chunk(i, 0))

    pltpu.sync_copy(args_hbm.at[pl.ds(base, B_PER_W)], args_v)

    pltpu.make_async_copy(
        table_hbm.at[pl.ds(0, B_PER_W)], rows_v, sem
    ).wait()

    col63 = lane * 0 + (D - 1)

    def fix(i, carry):
        b16 = pl.multiple_of(i * LANES, LANES)
        t = idx_v[pl.ds(b16, LANES)]
        a = args_v[pl.ds(b16, LANES)]
        m = (t == INT_PO) | (t == INT_NE)
        plsc.store_scatter(rows_v, [b16 + lane, col63], a, mask=m)
        return carry

    lax.fori_loop(0, N_CHUNKS, fix, 0)

    pltpu.sync_copy(rows_v, out_hbm.at[pl.ds(base, B_PER_W)])


def kernel(node_types, node_args, table):
    out = _build_sc_embed()(node_types, node_args, table)
    return out.reshape(N, 1, D)


# speedup vs baseline: 1.0036x; 1.0008x over previous
import functools

import jax
import jax.numpy as jnp
from jax import lax
from jax.experimental import pallas as pl
from jax.experimental.pallas import tpu as pltpu
from jax.experimental.pallas import tpu_sc as plsc

N = 16384
D = 64
NUM_CORES = 2
NUM_SUBCORES = 16
NUM_WORKERS = NUM_CORES * NUM_SUBCORES
B_PER_W = N // NUM_WORKERS
LANES = 16
N_CHUNKS = B_PER_W // LANES
INT_PO = 1
INT_NE = 2


@functools.cache
def _build_sc_embed():
    mesh = plsc.VectorSubcoreMesh(
        core_axis_name="c",
        subcore_axis_name="s",
        num_cores=NUM_CORES,
        num_subcores=NUM_SUBCORES,
    )
    return pl.kernel(
        _sc_embed_body,
        out_type=jax.ShapeDtypeStruct((N, D), jnp.float32),
        mesh=mesh,
        compiler_params=pltpu.CompilerParams(
            use_tc_tiling_on_sc=True, needs_layout_passes=False
        ),
        scratch_types=[
            pltpu.VMEM((B_PER_W,), jnp.int32),
            pltpu.VMEM((B_PER_W,), jnp.float32),
            pltpu.VMEM((B_PER_W, D), jnp.float32),
            pltpu.SemaphoreType.DMA,
        ],
    )


def _sc_embed_body(
    types_hbm, args_hbm, table_hbm, out_hbm, idx_v, args_v, rows_v, sem
):
    wid = lax.axis_index("s") * NUM_CORES + lax.axis_index("c")
    base = wid * B_PER_W

    pltpu.sync_copy(types_hbm.at[pl.ds(base, B_PER_W)], idx_v)

    lane = lax.iota(jnp.int32, LANES)

    def chunk(i, carry):
        b16 = pl.multiple_of(i * LANES, LANES)
        t = idx_v[pl.ds(b16, LANES)]

        for l in range(LANES):
            row = jnp.max(jnp.where(lane == l, t, 0))
            pltpu.async_copy(
                table_hbm.at[pl.ds(row, 1)],
                rows_v.at[pl.ds(b16 + l, 1)],
                sem,
            )
        return carry

    lax.fori_loop(0, N_CHUNKS, chunk, 0)

    pltpu.sync_copy(args_hbm.at[pl.ds(base, B_PER_W)], args_v)

    pltpu.make_async_copy(
        table_hbm.at[pl.ds(0, B_PER_W)], rows_v, sem
    ).wait()

    col63 = lane * 0 + (D - 1)

    def fix(i, carry):
        b16 = pl.multiple_of(i * LANES, LANES)
        t = idx_v[pl.ds(b16, LANES)]
        a = args_v[pl.ds(b16, LANES)]
        m = (t == INT_PO) | (t == INT_NE)
        plsc.store_scatter(rows_v, [b16 + lane, col63], a, mask=m)
        return carry

    lax.fori_loop(0, N_CHUNKS, fix, 0)

    pltpu.sync_copy(rows_v, out_hbm.at[pl.ds(base, B_PER_W)])


def kernel(node_types, node_args, table):
    out = _build_sc_embed()(node_types, node_args, table)
    return out.reshape(N, 1, D)
